# baseline (speedup 1.0000x reference)
import functools

import jax
import jax.numpy as jnp
from jax import lax
from jax.experimental import pallas as pl
from jax.experimental.pallas import tpu as pltpu
from jax.experimental.pallas import tpu_sc as plsc

DIM = 64
L = 16
CH = 128
NW = 32
ROWS = 4096 * 50
RPW = ROWS // NW
NCH = RPW // CH
GRP = CH // L
EPS = 1e-6


def _ln_chunk(rows_v, out_v, a2k, b2k):
    def group(g, carry):
        base = g * L
        rid = base + lax.iota(jnp.int32, L)
        acc = jnp.zeros((L,), jnp.float32)
        acc2 = jnp.zeros((L,), jnp.float32)
        for d in range(DIM):
            cid = jnp.full((L,), d, jnp.int32)
            v = plsc.load_gather(rows_v, [rid, cid])
            acc = acc + v
            acc2 = acc2 + v * v
        mean = acc * (1.0 / DIM)
        var = (acc2 - acc * mean) * (1.0 / (DIM - 1))
        var = jnp.maximum(var, 0.0)
        y = plsc.bitcast(
            jnp.int32(0x5F3759DF) - (plsc.bitcast(var, jnp.int32) >> 1),
            jnp.float32,
        )
        for _ in range(3):
            y = y * (1.5 - 0.5 * var * y * y)
        inv = 1.0 / (var * y + EPS)
        dnums = lax.GatherDimensionNumbers(
            offset_dims=(), collapsed_slice_dims=(0,), start_index_map=(0,))
        for r in range(L):
            rr = jnp.full((L, 1), r, jnp.int32)
            mb = lax.gather(mean, rr, dnums, (1,),
                            mode=lax.GatherScatterMode.PROMISE_IN_BOUNDS)
            ib = lax.gather(inv, rr, dnums, (1,),
                            mode=lax.GatherScatterMode.PROMISE_IN_BOUNDS)
            row = base + r
            for k in range(DIM // L):
                x = rows_v[row, pl.ds(k * L, L)]
                out_v[row, pl.ds(k * L, L)] = (x - mb) * ib * a2k[k] + b2k[k]
        return carry

    lax.fori_loop(0, GRP, group, 0)


def _body(inp_hbm, table_hbm, a2_hbm, b2_hbm, out_hbm,
          idx_v, rows_v, out_v, a2_v, b2_v, sem):
    wid = lax.axis_index("s") * 2 + lax.axis_index("c")
    pltpu.sync_copy(inp_hbm.at[wid], idx_v)
    pltpu.sync_copy(a2_hbm, a2_v)
    pltpu.sync_copy(b2_hbm, b2_v)
    a2k = [a2_v[pl.ds(k * L, L)] for k in range(DIM // L)]
    b2k = [b2_v[pl.ds(k * L, L)] for k in range(DIM // L)]

    def chunk(c, carry):
        pltpu.async_copy(table_hbm.at[idx_v.at[c]], rows_v, sem).wait()
        _ln_chunk(rows_v, out_v, a2k, b2k)
        pltpu.sync_copy(out_v, out_hbm.at[pl.ds(wid * RPW + c * CH, CH)])
        return carry

    lax.fori_loop(0, NCH, chunk, 0)


def kernel(inp, table, a_2, b_2):
    b, s = inp.shape
    inp_r = inp.reshape(NW, NCH, CH)
    mesh = plsc.VectorSubcoreMesh(core_axis_name="c", subcore_axis_name="s")
    run = functools.partial(
        pl.kernel,
        out_type=jax.ShapeDtypeStruct((ROWS, DIM), jnp.float32),
        mesh=mesh,
        compiler_params=pltpu.CompilerParams(
            needs_layout_passes=False, use_tc_tiling_on_sc=False),
        scratch_types=[
            pltpu.VMEM((NCH, CH), jnp.int32),
            pltpu.VMEM((CH, DIM), jnp.float32),
            pltpu.VMEM((CH, DIM), jnp.float32),
            pltpu.VMEM((DIM,), jnp.float32),
            pltpu.VMEM((DIM,), jnp.float32),
            pltpu.SemaphoreType.DMA,
        ],
    )(_body)
    out = run(inp_r, table, a_2, b_2)
    return out.reshape(b, s, DIM)
